# Initial kernel scaffold; baseline (speedup 1.0000x reference)
import jax
import jax.numpy as jnp
from jax.experimental import pallas as pl


def kernel(x, edge_index, batch, W1, a_src1, a_dst1, b1, bn1_g, bn1_b, Wg, bg, bn2_g, bn2_b, Wl, bl, Wr, bn3_g, bn3_b, W2, a_src2, a_dst2, b2, bn4_g, bn4_b, P1, pb1, P2, pb2):
    raise NotImplementedError("write your pallas kernel here")


# speedup vs baseline: 1.0825x; 1.0825x over previous
import functools
import jax
import jax.numpy as jnp
from jax.experimental import pallas as pl
from jax.experimental.pallas import tpu as pltpu

_N = 50000
_G = 1024
_HID = 128
_EPS = 1e-5
_BLK = 1000


def _elu(v):
    return jnp.where(v > 0, v, jnp.exp(jnp.minimum(v, 0.0)) - 1.0)


def _dense_call(body, out_dims, ins):
    grid = _N // _BLK
    in_specs = []
    for a, blocked in ins:
        if blocked:
            in_specs.append(
                pl.BlockSpec((_BLK, a.shape[1]), lambda i: (i, 0)))
        else:
            in_specs.append(
                pl.BlockSpec(a.shape, functools.partial(
                    lambda nd, i: (0,) * nd, a.ndim)))
    out_specs = [pl.BlockSpec((_BLK, d), lambda i: (i, 0)) for d in out_dims]
    out_shape = [jax.ShapeDtypeStruct((_N, d), jnp.float32) for d in out_dims]
    if len(out_dims) == 1:
        out_specs, out_shape = out_specs[0], out_shape[0]
    return pl.pallas_call(
        body, grid=(grid,), in_specs=in_specs, out_specs=out_specs,
        out_shape=out_shape)(*[a for a, _ in ins])


def _s1_body(x_r, w_r, asr_r, adr_r, h_r, a_r, b_r):
    h = jnp.dot(x_r[...], w_r[...], preferred_element_type=jnp.float32)
    h_r[...] = h
    a_r[...] = jnp.dot(h, asr_r[...], preferred_element_type=jnp.float32)
    b_r[...] = jnp.dot(h, adr_r[...], preferred_element_type=jnp.float32)


def _s2_body(g_r, sc_r, sh_r, w_r, o_r):
    z = _elu(g_r[...] * sc_r[...] + sh_r[...])
    o_r[...] = jnp.dot(z, w_r[...], preferred_element_type=jnp.float32)


def _s3_body(g_r, sc_r, sh_r, wr_r, z_r, zr_r):
    z = _elu(g_r[...] * sc_r[...] + sh_r[...])
    z_r[...] = z
    zr_r[...] = jnp.dot(z, wr_r[...], preferred_element_type=jnp.float32)


def _s4_body(agg_r, zr_r, wl_r, sc_r, sh_r, w2_r, asr_r, adr_r,
             h_r, a_r, b_r):
    sage = jnp.dot(agg_r[...], wl_r[...],
                   preferred_element_type=jnp.float32) + zr_r[...]
    z3 = _elu(sage * sc_r[...] + sh_r[...])
    h = jnp.dot(z3, w2_r[...], preferred_element_type=jnp.float32)
    h_r[...] = h
    a_r[...] = jnp.dot(h, asr_r[...], preferred_element_type=jnp.float32)
    b_r[...] = jnp.dot(h, adr_r[...], preferred_element_type=jnp.float32)


def _s5_body(g_r, sc_r, sh_r, o_r):
    o_r[...] = _elu(g_r[...] * sc_r[...] + sh_r[...])


def _s6_body(sums_r, cnt_r, p1_r, pb1_r, p2_r, pb2_r, gr_r, pr_r):
    graph = sums_r[...] * (1.0 / jnp.maximum(cnt_r[...], 1.0))
    gr_r[...] = graph
    t = jax.nn.relu(
        jnp.dot(graph, p1_r[...], preferred_element_type=jnp.float32)
        + pb1_r[...])
    pr_r[...] = jnp.dot(t, p2_r[...],
                        preferred_element_type=jnp.float32) + pb2_r[...]


def _blockdiag(a):
    heads, ch = a.shape
    eye = jnp.eye(heads, dtype=a.dtype)
    return (a[:, :, None] * eye[:, None, :]).reshape(heads * ch, heads)


def kernel(x, edge_index, batch, W1, a_src1, a_dst1, b1, bn1_g, bn1_b,
           Wg, bg, bn2_g, bn2_b, Wl, bl, Wr, bn3_g, bn3_b,
           W2, a_src2, a_dst2, b2, bn4_g, bn4_b, P1, pb1, P2, pb2):
    n = _N
    loop = jnp.arange(n, dtype=edge_index.dtype)
    src0, dst0 = edge_index[0], edge_index[1]
    src = jnp.concatenate([src0, loop])
    dst = jnp.concatenate([dst0, loop])
    bns = 1.0 / jnp.sqrt(jnp.float32(1.0 + _EPS))

    As1 = _blockdiag(a_src1)
    Ad1 = _blockdiag(a_dst1)
    h1, as1, ad1 = _dense_call(
        _s1_body, [8 * _HID, 8, 8],
        [(x, True), (W1, False), (As1, False), (Ad1, False)])

    ex = jnp.exp(jax.nn.leaky_relu(as1[src] + ad1[dst], 0.2))
    denom = jax.ops.segment_sum(ex, dst, num_segments=n)
    msg = jax.ops.segment_sum(
        h1.reshape(n, 8, _HID)[src] * ex[:, :, None], dst, num_segments=n)
    gat1 = (msg / (denom + 1e-16)[:, :, None]).reshape(n, 8 * _HID)

    sc1 = (bn1_g * bns)[None, :]
    sh1 = (b1 * bn1_g * bns + bn1_b)[None, :]
    h2 = _dense_call(_s2_body, [_HID],
                     [(gat1, True), (sc1, False), (sh1, False), (Wg, False)])

    ones_e = jnp.ones(src.shape[0], dtype=jnp.float32)
    deg = jax.ops.segment_sum(ones_e, dst, num_segments=n)
    dinv = jnp.where(deg > 0, deg ** -0.5, 0.0)
    coef = dinv[src] * dinv[dst]
    gcn = jax.ops.segment_sum(h2[src] * coef[:, None], dst, num_segments=n)

    sc2 = (bn2_g * bns)[None, :]
    sh2 = (bg * bn2_g * bns + bn2_b)[None, :]
    z2, zr = _dense_call(
        _s3_body, [_HID, _HID],
        [(gcn, True), (sc2, False), (sh2, False), (Wr, False)])

    cnt = jax.ops.segment_sum(jnp.ones(src0.shape[0], dtype=jnp.float32),
                              dst0, num_segments=n)
    agg = jax.ops.segment_sum(z2[src0], dst0, num_segments=n)
    agg = agg / jnp.maximum(cnt, 1.0)[:, None]

    sc3 = (bn3_g * bns)[None, :]
    sh3 = ((bl + 0.0) * bn3_g * bns + bn3_b)[None, :]
    As2 = _blockdiag(a_src2)
    Ad2 = _blockdiag(a_dst2)
    h4, as2, ad2 = _dense_call(
        _s4_body, [4 * _HID, 4, 4],
        [(agg, True), (zr, True), (Wl, False), (sc3, False), (sh3, False),
         (W2, False), (As2, False), (Ad2, False)])

    ex2 = jnp.exp(jax.nn.leaky_relu(as2[src] + ad2[dst], 0.2))
    denom2 = jax.ops.segment_sum(ex2, dst, num_segments=n)
    msg2 = jax.ops.segment_sum(
        h4.reshape(n, 4, _HID)[src] * ex2[:, :, None], dst, num_segments=n)
    gat2 = (msg2 / (denom2 + 1e-16)[:, :, None]).mean(axis=1)

    sc4 = (bn4_g * bns)[None, :]
    sh4 = (b2 * bn4_g * bns + bn4_b)[None, :]
    node_repr = _dense_call(
        _s5_body, [_HID], [(gat2, True), (sc4, False), (sh4, False)])

    sums = jax.ops.segment_sum(node_repr, batch, num_segments=_G)
    cntg = jax.ops.segment_sum(jnp.ones((n,), jnp.float32), batch,
                               num_segments=_G)
    graph_repr, proj_repr = pl.pallas_call(
        _s6_body,
        in_specs=[pl.BlockSpec((_G, _HID), lambda: (0, 0)),
                  pl.BlockSpec((_G, 1), lambda: (0, 0)),
                  pl.BlockSpec((_HID, _HID), lambda: (0, 0)),
                  pl.BlockSpec((1, _HID), lambda: (0, 0)),
                  pl.BlockSpec((_HID, 64), lambda: (0, 0)),
                  pl.BlockSpec((1, 64), lambda: (0, 0))],
        out_specs=[pl.BlockSpec((_G, _HID), lambda: (0, 0)),
                   pl.BlockSpec((_G, 64), lambda: (0, 0))],
        out_shape=[jax.ShapeDtypeStruct((_G, _HID), jnp.float32),
                   jax.ShapeDtypeStruct((_G, 64), jnp.float32)],
    )(sums, cntg[:, None], P1, pb1[None, :], P2, pb2[None, :])

    return (node_repr, graph_repr, proj_repr)
